# Initial kernel scaffold; baseline (speedup 1.0000x reference)
import jax
import jax.numpy as jnp
from jax.experimental import pallas as pl


def kernel(xyz, points, W_emb1, b_emb1, W_emb2, b_emb2, Wq, bq, Wk, bk, Wv, bv, Wo, bo, ln1_g, ln1_b, W1, b1f, W2, b2f, ln2_g, ln2_b, W_out, b_out):
    raise NotImplementedError("write your pallas kernel here")


# speedup vs baseline: 1.1510x; 1.1510x over previous
import functools

import jax
import jax.numpy as jnp
import numpy as np
from jax.experimental import pallas as pl
from jax.experimental.pallas import tpu as pltpu

NPOINT = 1024
NSAMPLE = 32
CH = 256
HEADS = 4
LAYERS = 4
HID = 128
FFN = 512
OUT = 4
IN_CH = 6
DH = CH // HEADS
SCALE = 1.0 / float(np.sqrt(DH))


def _emb_body(p_ref, w1_ref, b1_ref, w2_ref, b2_ref, o_ref):
    h = jnp.dot(p_ref[...], w1_ref[...], preferred_element_type=jnp.float32)
    h = jnp.maximum(h + b1_ref[...], 0.0)
    o = jnp.dot(h, w2_ref[...], preferred_element_type=jnp.float32)
    o_ref[...] = jnp.maximum(o + b2_ref[...], 0.0)


def _embed(points2d, W1, b1, W2, b2):
    M = points2d.shape[0]
    TILE = 2048
    return pl.pallas_call(
        _emb_body,
        grid=(M // TILE,),
        in_specs=[
            pl.BlockSpec((TILE, IN_CH), lambda i: (i, 0)),
            pl.BlockSpec((IN_CH, HID), lambda i: (0, 0)),
            pl.BlockSpec((1, HID), lambda i: (0, 0)),
            pl.BlockSpec((HID, CH), lambda i: (0, 0)),
            pl.BlockSpec((1, CH), lambda i: (0, 0)),
        ],
        out_specs=pl.BlockSpec((TILE, CH), lambda i: (i, 0)),
        out_shape=jax.ShapeDtypeStruct((M, CH), jnp.float32),
    )(points2d, W1, b1.reshape(1, HID), W2, b2.reshape(1, CH))


def _dist_body(q_ref, x_ref, o_ref):
    q = q_ref[0]
    x = x_ref[0]
    nq = jnp.sum(q * q, axis=-1, keepdims=True)
    nx = jnp.sum(x * x, axis=-1)[None, :]
    qx = jax.lax.dot_general(q, x, (((1,), (1,)), ((), ())),
                             preferred_element_type=jnp.float32)
    o_ref[0] = nq + nx - 2.0 * qx


def _distances(new_xyz, xyz):
    B, N, _ = xyz.shape
    KT = 2048
    return pl.pallas_call(
        _dist_body,
        grid=(B, N // KT),
        in_specs=[
            pl.BlockSpec((1, NPOINT, 3), lambda b, j: (b, 0, 0)),
            pl.BlockSpec((1, KT, 3), lambda b, j: (b, j, 0)),
        ],
        out_specs=pl.BlockSpec((1, NPOINT, KT), lambda b, j: (b, 0, j)),
        out_shape=jax.ShapeDtypeStruct((B, NPOINT, N), jnp.float32),
    )(new_xyz, xyz)


def _layer_norm(x, g, b):
    m = jnp.mean(x, axis=-1, keepdims=True)
    v = jnp.mean((x - m) ** 2, axis=-1, keepdims=True)
    return (x - m) / jnp.sqrt(v + 1e-5) * g + b


def _tx_body(x_ref, wq, bq, wk, bk, wv, bv, wo, bo, g1, be1,
             w1, b1f, w2, b2f, g2, be2, wout, bout, o_ref):
    x = x_ref[0]
    for l in range(LAYERS):
        q = jnp.dot(x, wq[l], preferred_element_type=jnp.float32) + bq[l][None, :]
        k = jnp.dot(x, wk[l], preferred_element_type=jnp.float32) + bk[l][None, :]
        v = jnp.dot(x, wv[l], preferred_element_type=jnp.float32) + bv[l][None, :]
        outs = []
        for h in range(HEADS):
            qh = q[:, h * DH:(h + 1) * DH]
            kh = k[:, h * DH:(h + 1) * DH]
            vh = v[:, h * DH:(h + 1) * DH]
            s = jax.lax.dot_general(qh, kh, (((1,), (1,)), ((), ())),
                                    preferred_element_type=jnp.float32) * SCALE
            s = s - jnp.max(s, axis=-1, keepdims=True)
            e = jnp.exp(s)
            a = e / jnp.sum(e, axis=-1, keepdims=True)
            outs.append(jnp.dot(a, vh, preferred_element_type=jnp.float32))
        o = jnp.concatenate(outs, axis=-1)
        x = _layer_norm(x + jnp.dot(o, wo[l], preferred_element_type=jnp.float32)
                        + bo[l][None, :], g1[l][None, :], be1[l][None, :])
        hdn = jnp.maximum(jnp.dot(x, w1[l], preferred_element_type=jnp.float32)
                          + b1f[l][None, :], 0.0)
        ffn = jnp.dot(hdn, w2[l], preferred_element_type=jnp.float32) + b2f[l][None, :]
        x = _layer_norm(x + ffn, g2[l][None, :], be2[l][None, :])
    o_ref[0] = jnp.dot(x, wout[...], preferred_element_type=jnp.float32) + bout[...]


def _transformer(x0, Wq, bq, Wk, bk, Wv, bv, Wo, bo, ln1_g, ln1_b,
                 W1, b1f, W2, b2f, ln2_g, ln2_b, W_out, b_out):
    B = x0.shape[0]
    full = lambda *s: pl.BlockSpec(s, lambda b: (0,) * len(s))
    return pl.pallas_call(
        _tx_body,
        grid=(B,),
        in_specs=[
            pl.BlockSpec((1, NPOINT, CH), lambda b: (b, 0, 0)),
            full(LAYERS, CH, CH), full(LAYERS, CH),
            full(LAYERS, CH, CH), full(LAYERS, CH),
            full(LAYERS, CH, CH), full(LAYERS, CH),
            full(LAYERS, CH, CH), full(LAYERS, CH),
            full(LAYERS, CH), full(LAYERS, CH),
            full(LAYERS, CH, FFN), full(LAYERS, FFN),
            full(LAYERS, FFN, CH), full(LAYERS, CH),
            full(LAYERS, CH), full(LAYERS, CH),
            full(CH, OUT), full(1, OUT),
        ],
        out_specs=pl.BlockSpec((1, NPOINT, OUT), lambda b: (b, 0, 0)),
        out_shape=jax.ShapeDtypeStruct((B, NPOINT, OUT), jnp.float32),
    )(x0, Wq, bq, Wk, bk, Wv, bv, Wo, bo, ln1_g, ln1_b,
      W1, b1f, W2, b2f, ln2_g, ln2_b, W_out, b_out.reshape(1, OUT))


def kernel(xyz, points, W_emb1, b_emb1, W_emb2, b_emb2, Wq, bq, Wk, bk,
           Wv, bv, Wo, bo, ln1_g, ln1_b, W1, b1f, W2, b2f, ln2_g, ln2_b,
           W_out, b_out):
    B, N, _ = xyz.shape
    feat = _embed(points.reshape(B * N, IN_CH), W_emb1, b_emb1,
                  W_emb2, b_emb2).reshape(B, N, CH)
    stride = N // NPOINT
    new_xyz = xyz[:, ::stride, :]
    d = _distances(new_xyz, xyz)

    idx = jax.lax.top_k(-d, NSAMPLE)[1]
    bi = jnp.arange(B)[:, None, None]
    grouped = feat[bi, idx]
    x0 = jnp.max(grouped, axis=2)

    out_s = _transformer(x0, Wq, bq, Wk, bk, Wv, bv, Wo, bo, ln1_g, ln1_b,
                         W1, b1f, W2, b2f, ln2_g, ln2_b, W_out, b_out)

    grouped_out = jnp.broadcast_to(out_s[:, :, None, :], (B, NPOINT, NSAMPLE, OUT))
    flat_idx = idx.reshape(B, -1)
    flat_g = grouped_out.reshape(B, -1, OUT)
    bi2 = jnp.arange(B)[:, None]
    recon = jnp.zeros((B, N, OUT), jnp.float32).at[bi2, flat_idx].add(flat_g)
    counts = jnp.zeros((B, N), jnp.float32).at[bi2, flat_idx].add(
        jnp.ones(flat_idx.shape, jnp.float32))
    counts = jnp.clip(counts, 1.0, None)
    return recon / counts[..., None]


# speedup vs baseline: 1.2018x; 1.0441x over previous
import functools

import jax
import jax.numpy as jnp
import numpy as np
from jax import lax
from jax.experimental import pallas as pl
from jax.experimental.pallas import tpu as pltpu
from jax.experimental.pallas import tpu_sc as plsc

NC = 2
NS = 16
LANES = 16

NPOINT = 1024
NSAMPLE = 32
CH = 256
HEADS = 4
LAYERS = 4
HID = 128
FFN = 512
OUT = 4
IN_CH = 6
DH = CH // HEADS
SCALE = 1.0 / float(np.sqrt(DH))


def _emb_body(p_ref, w1_ref, b1_ref, w2_ref, b2_ref, o_ref):
    h = jnp.dot(p_ref[...], w1_ref[...], preferred_element_type=jnp.float32)
    h = jnp.maximum(h + b1_ref[...], 0.0)
    o = jnp.dot(h, w2_ref[...], preferred_element_type=jnp.float32)
    o_ref[...] = jnp.maximum(o + b2_ref[...], 0.0)


def _embed(points2d, W1, b1, W2, b2):
    M = points2d.shape[0]
    TILE = 2048
    return pl.pallas_call(
        _emb_body,
        grid=(M // TILE,),
        in_specs=[
            pl.BlockSpec((TILE, IN_CH), lambda i: (i, 0)),
            pl.BlockSpec((IN_CH, HID), lambda i: (0, 0)),
            pl.BlockSpec((1, HID), lambda i: (0, 0)),
            pl.BlockSpec((HID, CH), lambda i: (0, 0)),
            pl.BlockSpec((1, CH), lambda i: (0, 0)),
        ],
        out_specs=pl.BlockSpec((TILE, CH), lambda i: (i, 0)),
        out_shape=jax.ShapeDtypeStruct((M, CH), jnp.float32),
    )(points2d, W1, b1.reshape(1, HID), W2, b2.reshape(1, CH))


def _dist_body(q_ref, x_ref, o_ref):
    q = q_ref[0]
    x = x_ref[0]
    nq = jnp.sum(q * q, axis=-1, keepdims=True)
    nx = jnp.sum(x * x, axis=-1)[None, :]
    qx = jax.lax.dot_general(q, x, (((1,), (1,)), ((), ())),
                             preferred_element_type=jnp.float32)
    o_ref[0] = nq + nx - 2.0 * qx


def _distances(new_xyz, xyz):
    B, N, _ = xyz.shape
    KT = 2048
    return pl.pallas_call(
        _dist_body,
        grid=(B, N // KT),
        in_specs=[
            pl.BlockSpec((1, NPOINT, 3), lambda b, j: (b, 0, 0)),
            pl.BlockSpec((1, KT, 3), lambda b, j: (b, j, 0)),
        ],
        out_specs=pl.BlockSpec((1, NPOINT, KT), lambda b, j: (b, 0, j)),
        out_shape=jax.ShapeDtypeStruct((B, NPOINT, N), jnp.float32),
    )(new_xyz, xyz)


def _layer_norm(x, g, b):
    m = jnp.mean(x, axis=-1, keepdims=True)
    v = jnp.mean((x - m) ** 2, axis=-1, keepdims=True)
    return (x - m) / jnp.sqrt(v + 1e-5) * g + b


def _tx_body(x_ref, wq, bq, wk, bk, wv, bv, wo, bo, g1, be1,
             w1, b1f, w2, b2f, g2, be2, wout, bout, o_ref):
    x = x_ref[0]
    for l in range(LAYERS):
        q = jnp.dot(x, wq[l], preferred_element_type=jnp.float32) + bq[l][None, :]
        k = jnp.dot(x, wk[l], preferred_element_type=jnp.float32) + bk[l][None, :]
        v = jnp.dot(x, wv[l], preferred_element_type=jnp.float32) + bv[l][None, :]
        outs = []
        for h in range(HEADS):
            qh = q[:, h * DH:(h + 1) * DH]
            kh = k[:, h * DH:(h + 1) * DH]
            vh = v[:, h * DH:(h + 1) * DH]
            s = jax.lax.dot_general(qh, kh, (((1,), (1,)), ((), ())),
                                    preferred_element_type=jnp.float32) * SCALE
            s = s - jnp.max(s, axis=-1, keepdims=True)
            e = jnp.exp(s)
            a = e / jnp.sum(e, axis=-1, keepdims=True)
            outs.append(jnp.dot(a, vh, preferred_element_type=jnp.float32))
        o = jnp.concatenate(outs, axis=-1)
        x = _layer_norm(x + jnp.dot(o, wo[l], preferred_element_type=jnp.float32)
                        + bo[l][None, :], g1[l][None, :], be1[l][None, :])
        hdn = jnp.maximum(jnp.dot(x, w1[l], preferred_element_type=jnp.float32)
                          + b1f[l][None, :], 0.0)
        ffn = jnp.dot(hdn, w2[l], preferred_element_type=jnp.float32) + b2f[l][None, :]
        x = _layer_norm(x + ffn, g2[l][None, :], be2[l][None, :])
    o_ref[0] = jnp.dot(x, wout[...], preferred_element_type=jnp.float32) + bout[...]


def _transformer(x0, Wq, bq, Wk, bk, Wv, bv, Wo, bo, ln1_g, ln1_b,
                 W1, b1f, W2, b2f, ln2_g, ln2_b, W_out, b_out):
    B = x0.shape[0]
    full = lambda *s: pl.BlockSpec(s, lambda b: (0,) * len(s))
    return pl.pallas_call(
        _tx_body,
        grid=(B,),
        in_specs=[
            pl.BlockSpec((1, NPOINT, CH), lambda b: (b, 0, 0)),
            full(LAYERS, CH, CH), full(LAYERS, CH),
            full(LAYERS, CH, CH), full(LAYERS, CH),
            full(LAYERS, CH, CH), full(LAYERS, CH),
            full(LAYERS, CH, CH), full(LAYERS, CH),
            full(LAYERS, CH), full(LAYERS, CH),
            full(LAYERS, CH, FFN), full(LAYERS, FFN),
            full(LAYERS, FFN, CH), full(LAYERS, CH),
            full(LAYERS, CH), full(LAYERS, CH),
            full(CH, OUT), full(1, OUT),
        ],
        out_specs=pl.BlockSpec((1, NPOINT, OUT), lambda b: (b, 0, 0)),
        out_shape=jax.ShapeDtypeStruct((B, NPOINT, OUT), jnp.float32),
    )(x0, Wq, bq, Wk, bk, Wv, bv, Wo, bo, ln1_g, ln1_b,
      W1, b1f, W2, b2f, ln2_g, ln2_b, W_out, b_out.reshape(1, OUT))


def _sc_scatter(idx, out_s, B, N):
    P_TILE = NPOINT // 8
    GROUPS = P_TILE // 4
    idx2 = (idx.reshape(B, NPOINT * NSAMPLE // 128, 128)
            + (jnp.arange(B, dtype=jnp.int32) % 2)[:, None, None] * N)
    idx2 = idx2.reshape(B * NPOINT * NSAMPLE // 128, 128)
    outs16 = jnp.concatenate(
        [out_s.reshape(B * NPOINT, OUT),
         jnp.ones((B * NPOINT, 1), jnp.float32),
         jnp.zeros((B * NPOINT, 11), jnp.float32)], axis=1)

    mesh = plsc.VectorSubcoreMesh(core_axis_name="c", subcore_axis_name="s",
                                  num_cores=NC, num_subcores=NS)

    @functools.partial(
        pl.kernel, mesh=mesh,
        out_type=jax.ShapeDtypeStruct((B * N, 16), jnp.float32),
        scratch_types=[
            pltpu.VMEM((GROUPS, 128), jnp.int32),
            pltpu.VMEM((P_TILE, 16), jnp.float32),
            pltpu.VMEM((128, 16), jnp.float32),
            pltpu.VMEM((2 * N // NS, 16), jnp.float32),
            pltpu.VMEM_SHARED((2 * N, 16), jnp.float32),
        ],
        compiler_params=pltpu.CompilerParams(use_tc_tiling_on_sc=False),
    )
    def k(idx_hbm, outs_hbm, out_hbm, idxg, outsbuf, upd, dbuf, acc):
        c = lax.axis_index("c")
        s = lax.axis_index("s")
        b = 2 * c + s // 8
        t = s % 8
        zrow = jnp.zeros((16,), jnp.float32)
        rows_per_tile = 2 * N // NS

        pltpu.sync_copy(idx_hbm.at[pl.ds(b * (NPOINT // 4) + t * GROUPS,
                                         GROUPS)], idxg)
        pltpu.sync_copy(outs_hbm.at[pl.ds(b * NPOINT + t * P_TILE, P_TILE)],
                        outsbuf)

        def zbody(i, _):
            dbuf[i] = zrow
            return 0
        lax.fori_loop(0, rows_per_tile, zbody, 0)
        pltpu.sync_copy(dbuf, acc.at[pl.ds(s * rows_per_tile, rows_per_tile)])
        plsc.subcore_barrier()

        def sbody(g, _):
            for q in range(4):
                o = outsbuf[4 * g + q]
                for j in range(NSAMPLE):
                    upd[q * NSAMPLE + j] = o
            pltpu.sync_copy(upd, acc.at[idxg.at[g]], add=True)
            return 0
        lax.fori_loop(0, GROUPS, sbody, 0)
        plsc.subcore_barrier()

        pltpu.sync_copy(acc.at[pl.ds(s * rows_per_tile, rows_per_tile)],
                        out_hbm.at[pl.ds(c * 2 * N + s * rows_per_tile,
                                         rows_per_tile)])

    raw = k(idx2, outs16)

    def _div_body(a_ref, o_ref):
        a = a_ref[...]
        o_ref[...] = a[:, :OUT] / jnp.maximum(a[:, OUT:OUT + 1], 1.0)

    out = pl.pallas_call(
        _div_body,
        grid=(8,),
        in_specs=[pl.BlockSpec((B * N // 8, 16), lambda i: (i, 0))],
        out_specs=pl.BlockSpec((B * N // 8, OUT), lambda i: (i, 0)),
        out_shape=jax.ShapeDtypeStruct((B * N, OUT), jnp.float32),
    )(raw)
    return out.reshape(B, N, OUT)


def kernel(xyz, points, W_emb1, b_emb1, W_emb2, b_emb2, Wq, bq, Wk, bk,
           Wv, bv, Wo, bo, ln1_g, ln1_b, W1, b1f, W2, b2f, ln2_g, ln2_b,
           W_out, b_out):
    B, N, _ = xyz.shape
    feat = _embed(points.reshape(B * N, IN_CH), W_emb1, b_emb1,
                  W_emb2, b_emb2).reshape(B, N, CH)
    stride = N // NPOINT
    new_xyz = xyz[:, ::stride, :]
    d = _distances(new_xyz, xyz)

    idx = jax.lax.top_k(-d, NSAMPLE)[1]
    bi = jnp.arange(B)[:, None, None]
    grouped = feat[bi, idx]
    x0 = jnp.max(grouped, axis=2)

    out_s = _transformer(x0, Wq, bq, Wk, bk, Wv, bv, Wo, bo, ln1_g, ln1_b,
                         W1, b1f, W2, b2f, ln2_g, ln2_b, W_out, b_out)

    return _sc_scatter(idx.astype(jnp.int32), out_s, B, N)
